# baseline (speedup 1.0000x reference)
import jax
import jax.numpy as jnp
from jax import lax
from jax.experimental import pallas as pl
from jax.experimental.pallas import tpu as pltpu
from jax.experimental.pallas import tpu_sc as plsc

_MAX_CTX = 32768
_NUM_CORES = 2
_NUM_SUBCORES = 16
_NUM_WORKERS = _NUM_CORES * _NUM_SUBCORES
_ROWS_PER_WORKER = _MAX_CTX // _NUM_WORKERS


def _copy_body(src_hbm, out_hbm):
    wid = lax.axis_index("s") * _NUM_CORES + lax.axis_index("c")
    base = wid * _ROWS_PER_WORKER
    pltpu.sync_copy(
        src_hbm.at[pl.ds(base, _ROWS_PER_WORKER)],
        out_hbm.at[pl.ds(base, _ROWS_PER_WORKER)],
    )


def kernel(h, mem):
    b, l, d = h.shape
    flat = h.reshape(b * l, d)
    new_mem = pl.kernel(
        _copy_body,
        out_type=jax.ShapeDtypeStruct((b * l, d), h.dtype),
        mesh=plsc.VectorSubcoreMesh(
            core_axis_name="c", subcore_axis_name="s"
        ),
    )(flat)
    return (h, new_mem)


# speedup vs baseline: 20.8211x; 20.8211x over previous
import jax
import jax.numpy as jnp
from jax import lax
from jax.experimental import pallas as pl
from jax.experimental.pallas import tpu as pltpu
from jax.experimental.pallas import tpu_sc as plsc

_MAX_CTX = 32768
_DIM = 1024
_NUM_CORES = 2
_NUM_SUBCORES = 16
_NUM_WORKERS = _NUM_CORES * _NUM_SUBCORES
_ROWS_PER_WORKER = _MAX_CTX // _NUM_WORKERS
_CH = 32
_NCH = _ROWS_PER_WORKER // _CH


def _copy_body(src_hbm, out_hbm, buf0, buf1, gs0, gs1, ss0, ss1):
    wid = lax.axis_index("s") * _NUM_CORES + lax.axis_index("c")
    base = wid * _ROWS_PER_WORKER
    bufs = (buf0, buf1)
    gsems = (gs0, gs1)
    ssems = (ss0, ss1)
    for i in range(_NCH):
        buf = bufs[i % 2]
        if i >= 2:
            pltpu.make_async_copy(
                buf, out_hbm.at[pl.ds(base + (i - 2) * _CH, _CH)], ssems[i % 2]
            ).wait()
        cp = pltpu.make_async_copy(
            src_hbm.at[pl.ds(base + i * _CH, _CH)], buf, gsems[i % 2]
        )
        cp.start()
        cp.wait()
        pltpu.make_async_copy(
            buf, out_hbm.at[pl.ds(base + i * _CH, _CH)], ssems[i % 2]
        ).start()
    for i in (_NCH - 2, _NCH - 1):
        pltpu.make_async_copy(
            bufs[i % 2], out_hbm.at[pl.ds(base + i * _CH, _CH)], ssems[i % 2]
        ).wait()


def kernel(h, mem):
    b, l, d = h.shape
    flat = h.reshape(b * l, d)
    new_mem = pl.kernel(
        _copy_body,
        out_type=jax.ShapeDtypeStruct((b * l, d), h.dtype),
        mesh=plsc.VectorSubcoreMesh(
            core_axis_name="c", subcore_axis_name="s"
        ),
        scratch_types=[
            pltpu.VMEM((_CH, _DIM), jnp.float32),
            pltpu.VMEM((_CH, _DIM), jnp.float32),
            pltpu.SemaphoreType.DMA,
            pltpu.SemaphoreType.DMA,
            pltpu.SemaphoreType.DMA,
            pltpu.SemaphoreType.DMA,
        ],
    )(flat)
    return (h, new_mem)


# speedup vs baseline: 21.7533x; 1.0448x over previous
import jax
import jax.numpy as jnp
from jax import lax
from jax.experimental import pallas as pl
from jax.experimental.pallas import tpu as pltpu
from jax.experimental.pallas import tpu_sc as plsc

_MAX_CTX = 32768
_DIM = 1024
_NUM_CORES = 2
_NUM_SUBCORES = 16
_NUM_WORKERS = _NUM_CORES * _NUM_SUBCORES
_ROWS_PER_WORKER = _MAX_CTX // _NUM_WORKERS
_CH = 32
_NCH = _ROWS_PER_WORKER // _CH


def _copy_body(src_hbm, out_hbm, buf0, buf1, gs0, gs1, ss0, ss1):
    wid = lax.axis_index("s") * _NUM_CORES + lax.axis_index("c")
    base = wid * _ROWS_PER_WORKER
    bufs = (buf0, buf1)
    gsems = (gs0, gs1)
    ssems = (ss0, ss1)
    for i in range(_NCH):
        buf = bufs[i % 2]
        if i >= 2:
            pltpu.make_async_copy(
                buf, out_hbm.at[pl.ds(base + (i - 2) * _CH, _CH)], ssems[i % 2]
            ).wait()
        cp = pltpu.make_async_copy(
            src_hbm.at[pl.ds(base + i * _CH, _CH)], buf, gsems[i % 2]
        )
        cp.start()
        cp.wait()
        pltpu.make_async_copy(
            buf, out_hbm.at[pl.ds(base + i * _CH, _CH)], ssems[i % 2]
        ).start()
    for i in (_NCH - 2, _NCH - 1):
        pltpu.make_async_copy(
            bufs[i % 2], out_hbm.at[pl.ds(base + i * _CH, _CH)], ssems[i % 2]
        ).wait()


def _tc_copy_body(src_ref, dst_ref):
    dst_ref[...] = src_ref[...]


def kernel(h, mem):
    b, l, d = h.shape
    flat = h.reshape(b * l, d)
    new_mem = pl.kernel(
        _copy_body,
        out_type=jax.ShapeDtypeStruct((b * l, d), h.dtype),
        mesh=plsc.VectorSubcoreMesh(
            core_axis_name="c", subcore_axis_name="s"
        ),
        scratch_types=[
            pltpu.VMEM((_CH, _DIM), jnp.float32),
            pltpu.VMEM((_CH, _DIM), jnp.float32),
            pltpu.SemaphoreType.DMA,
            pltpu.SemaphoreType.DMA,
            pltpu.SemaphoreType.DMA,
            pltpu.SemaphoreType.DMA,
        ],
    )(flat)
    h_out = pl.pallas_call(
        _tc_copy_body,
        grid=(b, l // 512),
        in_specs=[pl.BlockSpec((1, 512, d), lambda i, j: (i, j, 0))],
        out_specs=pl.BlockSpec((1, 512, d), lambda i, j: (i, j, 0)),
        out_shape=jax.ShapeDtypeStruct((b, l, d), h.dtype),
    )(h)
    return (h_out, new_mem)


# speedup vs baseline: 25.9427x; 1.1926x over previous
import jax
import jax.numpy as jnp
from jax import lax
from jax.experimental import pallas as pl
from jax.experimental.pallas import tpu as pltpu
from jax.experimental.pallas import tpu_sc as plsc

_MAX_CTX = 32768
_DIM = 1024
_B, _L = 4, 8192
_NUM_CORES = 2
_NUM_SUBCORES = 16
_NUM_WORKERS = _NUM_CORES * _NUM_SUBCORES

_SPLIT = _MAX_CTX // 2
_SC_ROWS_PER_WORKER = (_MAX_CTX - _SPLIT) // _NUM_WORKERS
_CH = 32
_NCH = _SC_ROWS_PER_WORKER // _CH

_TC_BLK = 512


def _sc_body(src_hbm, out1_hbm, out2_hbm, buf0, buf1,
             gs0, gs1, s1a, s1b, s2a, s2b):
    wid = lax.axis_index("s") * _NUM_CORES + lax.axis_index("c")
    base = _SPLIT + wid * _SC_ROWS_PER_WORKER
    bidx = base // _L
    loff = base % _L
    bufs = (buf0, buf1)
    gsems = (gs0, gs1)
    s1 = (s1a, s1b)
    s2 = (s2a, s2b)
    for i in range(_NCH):
        k = i % 2
        buf = bufs[k]
        if i >= 2:
            prev = (i - 2) * _CH
            pltpu.make_async_copy(
                buf, out1_hbm.at[bidx, pl.ds(loff + prev, _CH)], s1[k]
            ).wait()
            pltpu.make_async_copy(
                buf, out2_hbm.at[pl.ds(base + prev, _CH)], s2[k]
            ).wait()
        cur = i * _CH
        cp = pltpu.make_async_copy(
            src_hbm.at[pl.ds(base + cur, _CH)], buf, gsems[k]
        )
        cp.start()
        cp.wait()
        pltpu.make_async_copy(
            buf, out1_hbm.at[bidx, pl.ds(loff + cur, _CH)], s1[k]
        ).start()
        pltpu.make_async_copy(
            buf, out2_hbm.at[pl.ds(base + cur, _CH)], s2[k]
        ).start()
    for i in (_NCH - 2, _NCH - 1):
        k = i % 2
        cur = i * _CH
        pltpu.make_async_copy(
            bufs[k], out1_hbm.at[bidx, pl.ds(loff + cur, _CH)], s1[k]
        ).wait()
        pltpu.make_async_copy(
            bufs[k], out2_hbm.at[pl.ds(base + cur, _CH)], s2[k]
        ).wait()


def _tc_body(h_ref, p1_ref, p2_ref, out1_ref, out2_ref):
    del p1_ref, p2_ref
    blk = h_ref[...]
    out1_ref[...] = blk
    out2_ref[...] = blk.reshape(_TC_BLK, _DIM)


def kernel(h, mem):
    b, l, d = h.shape
    flat = h.reshape(b * l, d)

    p1, p2 = pl.kernel(
        _sc_body,
        out_type=(
            jax.ShapeDtypeStruct((b, l, d), h.dtype),
            jax.ShapeDtypeStruct((b * l, d), h.dtype),
        ),
        mesh=plsc.VectorSubcoreMesh(
            core_axis_name="c", subcore_axis_name="s"
        ),
        scratch_types=[
            pltpu.VMEM((_CH, _DIM), jnp.float32),
            pltpu.VMEM((_CH, _DIM), jnp.float32),
            pltpu.SemaphoreType.DMA,
            pltpu.SemaphoreType.DMA,
            pltpu.SemaphoreType.DMA,
            pltpu.SemaphoreType.DMA,
            pltpu.SemaphoreType.DMA,
            pltpu.SemaphoreType.DMA,
        ],
    )(flat)

    n_batches = _SPLIT // l
    out1, out2 = pl.pallas_call(
        _tc_body,
        grid=(n_batches, l // _TC_BLK),
        in_specs=[
            pl.BlockSpec((1, _TC_BLK, d), lambda i, j: (i, j, 0)),
            pl.BlockSpec(memory_space=pl.ANY),
            pl.BlockSpec(memory_space=pl.ANY),
        ],
        out_specs=[
            pl.BlockSpec((1, _TC_BLK, d), lambda i, j: (i, j, 0)),
            pl.BlockSpec((_TC_BLK, d), lambda i, j: (i * (_L // _TC_BLK) + j, 0)),
        ],
        out_shape=[
            jax.ShapeDtypeStruct((b, l, d), h.dtype),
            jax.ShapeDtypeStruct((b * l, d), h.dtype),
        ],
        input_output_aliases={1: 0, 2: 1},
    )(h, p1, p2)
    return (out1, out2)


# speedup vs baseline: 27.6382x; 1.0654x over previous
import jax
import jax.numpy as jnp
from jax import lax
from jax.experimental import pallas as pl
from jax.experimental.pallas import tpu as pltpu
from jax.experimental.pallas import tpu_sc as plsc

_MAX_CTX = 32768
_DIM = 1024
_B, _L = 4, 8192
_NUM_CORES = 2
_NUM_SUBCORES = 16
_NUM_WORKERS = _NUM_CORES * _NUM_SUBCORES

_SPLIT = 24576
_SC_ROWS_PER_WORKER = (_MAX_CTX - _SPLIT) // _NUM_WORKERS
_CH = 32
_NCH = _SC_ROWS_PER_WORKER // _CH

_TC_BLK = 1024


def _sc_body(src_hbm, out1_hbm, out2_hbm, buf0, buf1,
             gs0, gs1, s1a, s1b, s2a, s2b):
    wid = lax.axis_index("s") * _NUM_CORES + lax.axis_index("c")
    base = _SPLIT + wid * _SC_ROWS_PER_WORKER
    bidx = base // _L
    loff = base % _L
    bufs = (buf0, buf1)
    gsems = (gs0, gs1)
    s1 = (s1a, s1b)
    s2 = (s2a, s2b)
    for i in range(_NCH):
        k = i % 2
        buf = bufs[k]
        if i >= 2:
            prev = (i - 2) * _CH
            pltpu.make_async_copy(
                buf, out1_hbm.at[bidx, pl.ds(loff + prev, _CH)], s1[k]
            ).wait()
            pltpu.make_async_copy(
                buf, out2_hbm.at[pl.ds(base + prev, _CH)], s2[k]
            ).wait()
        cur = i * _CH
        cp = pltpu.make_async_copy(
            src_hbm.at[pl.ds(base + cur, _CH)], buf, gsems[k]
        )
        cp.start()
        cp.wait()
        pltpu.make_async_copy(
            buf, out1_hbm.at[bidx, pl.ds(loff + cur, _CH)], s1[k]
        ).start()
        pltpu.make_async_copy(
            buf, out2_hbm.at[pl.ds(base + cur, _CH)], s2[k]
        ).start()
    for i in (_NCH - 2, _NCH - 1):
        k = i % 2
        cur = i * _CH
        pltpu.make_async_copy(
            bufs[k], out1_hbm.at[bidx, pl.ds(loff + cur, _CH)], s1[k]
        ).wait()
        pltpu.make_async_copy(
            bufs[k], out2_hbm.at[pl.ds(base + cur, _CH)], s2[k]
        ).wait()


def _tc_body(h_ref, p1_ref, p2_ref, out1_ref, out2_ref):
    del p1_ref, p2_ref
    blk = h_ref[...]
    out1_ref[...] = blk
    out2_ref[...] = blk.reshape(_TC_BLK, _DIM)


def kernel(h, mem):
    b, l, d = h.shape
    flat = h.reshape(b * l, d)

    p1, p2 = pl.kernel(
        _sc_body,
        out_type=(
            jax.ShapeDtypeStruct((b, l, d), h.dtype),
            jax.ShapeDtypeStruct((b * l, d), h.dtype),
        ),
        mesh=plsc.VectorSubcoreMesh(
            core_axis_name="c", subcore_axis_name="s"
        ),
        scratch_types=[
            pltpu.VMEM((_CH, _DIM), jnp.float32),
            pltpu.VMEM((_CH, _DIM), jnp.float32),
            pltpu.SemaphoreType.DMA,
            pltpu.SemaphoreType.DMA,
            pltpu.SemaphoreType.DMA,
            pltpu.SemaphoreType.DMA,
            pltpu.SemaphoreType.DMA,
            pltpu.SemaphoreType.DMA,
        ],
    )(flat)

    n_batches = _SPLIT // l
    out1, out2 = pl.pallas_call(
        _tc_body,
        grid=(n_batches, l // _TC_BLK),
        in_specs=[
            pl.BlockSpec((1, _TC_BLK, d), lambda i, j: (i, j, 0)),
            pl.BlockSpec(memory_space=pl.ANY),
            pl.BlockSpec(memory_space=pl.ANY),
        ],
        out_specs=[
            pl.BlockSpec((1, _TC_BLK, d), lambda i, j: (i, j, 0)),
            pl.BlockSpec((_TC_BLK, d), lambda i, j: (i * (_L // _TC_BLK) + j, 0)),
        ],
        out_shape=[
            jax.ShapeDtypeStruct((b, l, d), h.dtype),
            jax.ShapeDtypeStruct((b * l, d), h.dtype),
        ],
        input_output_aliases={1: 0, 2: 1},
    )(h, p1, p2)
    return (out1, out2)
